# baseline (speedup 1.0000x reference)
import functools

import jax
import jax.numpy as jnp
from jax import lax
from jax.experimental import pallas as pl
from jax.experimental.pallas import tpu as pltpu
from jax.experimental.pallas import tpu_sc as plsc

_L = 8
_F = 2
_T = 2 ** 19
_MASK = _T - 1
_RES = (16.0, 24.0, 36.0, 54.0, 81.0, 121.0, 182.0, 273.0)
_P2 = -1640531535
_C3 = 36
_EW = 88
_CH = 64
_NC = 2
_NS = 16
_NW = _NC * _NS


@functools.lru_cache(maxsize=None)
def _build_sc_embed(N):
    ppw = N // _NW
    nch = ppw // _CH
    mesh = plsc.VectorSubcoreMesh(core_axis_name="c", subcore_axis_name="s")

    @functools.partial(
        pl.kernel,
        mesh=mesh,
        out_type=jax.ShapeDtypeStruct((N // _CH, _EW, _CH), jnp.float32),
        scratch_types=[
            pltpu.VMEM((3, _CH), jnp.float32),
            pltpu.VMEM((96, _CH), jnp.int32),
            pltpu.VMEM((96, _CH), jnp.int32),
            pltpu.VMEM((8, _CH), jnp.int32),
            pltpu.VMEM((96, _CH), jnp.float32),
            pltpu.VMEM((8, _CH), jnp.float32),
            pltpu.VMEM((96 * _CH, 8), jnp.float32),
            pltpu.VMEM((8 * _CH, 48), jnp.float32),
            pltpu.VMEM((_EW, _CH), jnp.float32),
            pltpu.SemaphoreType.DMA,
        ],
        compiler_params=pltpu.CompilerParams(
            needs_layout_passes=False, use_tc_tiling_on_sc=False),
    )
    def sc_embed(ac_hbm, txy_hbm, tyt_hbm, txt_hbm, grid_hbm, out_hbm,
                 ct, idx_tab, idx_win, idx_grid, w_tab, w_grid,
                 g_tab, g_grid, emb, sem):
        wid = lax.axis_index("s") * _NC + lax.axis_index("c")
        lane = lax.iota(jnp.int32, 16)

        def ifloor(pa):
            r = pa.astype(jnp.int32)
            return r - (r.astype(jnp.float32) > pa).astype(jnp.int32)

        def chunk(ci, carry):
            base = wid * ppw + ci * _CH
            pltpu.sync_copy(ac_hbm.at[:, pl.ds(base, _CH)], ct)

            def phase_idx(g, c_):
                s = g * 16
                x = ct[0, pl.ds(s, 16)]
                y = ct[1, pl.ds(s, 16)]
                z = ct[2, pl.ds(s, 16)]
                planes = ((y, z), (x, z), (x, y))
                for p in range(3):
                    a, bb = planes[p]
                    for l in range(_L):
                        res = _RES[l]
                        pa = a * res
                        pb = bb * res
                        fa = ifloor(pa)
                        fb = ifloor(pb)
                        wa = pa - fa.astype(jnp.float32)
                        wb = pb - fb.astype(jnp.float32)
                        hb0 = fb * _P2
                        hb1 = hb0 + _P2
                        fa1 = fa + 1
                        off = l * _T
                        j0 = p * 32 + l * 4
                        i00 = ((fa ^ hb0) & _MASK) + off
                        i01 = ((fa ^ hb1) & _MASK) + off
                        i10 = ((fa1 ^ hb0) & _MASK) + off
                        i11 = ((fa1 ^ hb1) & _MASK) + off
                        idx_tab[j0 + 0, pl.ds(s, 16)] = i00
                        idx_tab[j0 + 1, pl.ds(s, 16)] = i01
                        idx_tab[j0 + 2, pl.ds(s, 16)] = i10
                        idx_tab[j0 + 3, pl.ds(s, 16)] = i11
                        idx_win[j0 + 0, pl.ds(s, 16)] = i00 >> 2
                        idx_win[j0 + 1, pl.ds(s, 16)] = i01 >> 2
                        idx_win[j0 + 2, pl.ds(s, 16)] = i10 >> 2
                        idx_win[j0 + 3, pl.ds(s, 16)] = i11 >> 2
                        ua = 1.0 - wa
                        ub = 1.0 - wb
                        w_tab[j0 + 0, pl.ds(s, 16)] = ua * ub
                        w_tab[j0 + 1, pl.ds(s, 16)] = ua * wb
                        w_tab[j0 + 2, pl.ds(s, 16)] = wa * ub
                        w_tab[j0 + 3, pl.ds(s, 16)] = wa * wb
                p0 = x * 31.0
                p1 = y * 63.0
                p2 = z * 63.0
                f0 = jnp.clip(ifloor(p0), 0, 30)
                f1 = jnp.clip(ifloor(p1), 0, 62)
                f2 = jnp.clip(ifloor(p2), 0, 62)
                w0 = p0 - f0.astype(jnp.float32)
                w1 = p1 - f1.astype(jnp.float32)
                w2 = p2 - f2.astype(jnp.float32)
                u0 = 1.0 - w0
                u1 = 1.0 - w1
                u2 = 1.0 - w2
                gb = (f0 * 64 + f1) * 64 + f2
                for c in range(8):
                    dt, dx, dy = (c >> 2) & 1, (c >> 1) & 1, c & 1
                    idx_grid[c, pl.ds(s, 16)] = gb + (dt * 4096 + dx * 64 + dy)
                    wt = (w0 if dt else u0) * (w1 if dx else u1) * (w2 if dy else u2)
                    w_grid[c, pl.ds(s, 16)] = wt
                return c_

            lax.fori_loop(0, _CH // 16, phase_idx, 0)

            cps = []
            for p, tab in enumerate((txy_hbm, tyt_hbm, txt_hbm)):
                for q in range(32):
                    j = p * 32 + q
                    cps.append(pltpu.async_copy(
                        tab.at[idx_win.at[j]],
                        g_tab.at[pl.ds(j * _CH, _CH), :], sem))
            for c in range(8):
                cps.append(pltpu.async_copy(
                    grid_hbm.at[idx_grid.at[c]],
                    g_grid.at[pl.ds(c * _CH, _CH), :], sem))
            for cp in cps:
                cp.wait()

            def phase_sum(g, c_):
                s = g * 16
                pt = s + lane
                zero16 = jnp.zeros((16,), jnp.int32)
                one16 = zero16 + 1
                for p in range(3):
                    for l in range(_L):
                        j0 = p * 32 + l * 4
                        col = p * 16 + l * 2
                        acc0 = jnp.zeros((16,), jnp.float32)
                        acc1 = jnp.zeros((16,), jnp.float32)
                        for c in range(4):
                            w = w_tab[j0 + c, pl.ds(s, 16)]
                            sub2 = (idx_tab[j0 + c, pl.ds(s, 16)] & 3) * 2
                            i0 = pt + ((j0 + c) * _CH)
                            v0 = plsc.load_gather(g_tab, [i0, sub2])
                            v1 = plsc.load_gather(g_tab, [i0, sub2 + 1])
                            acc0 = acc0 + w * v0
                            acc1 = acc1 + w * v1
                        emb[col, pl.ds(s, 16)] = acc0
                        emb[col + 1, pl.ds(s, 16)] = acc1
                zf = jnp.zeros((16,), jnp.float32)
                for cc in range(84, _EW):
                    emb[cc, pl.ds(s, 16)] = zf
                wg = [w_grid[c, pl.ds(s, 16)] for c in range(8)]
                for ch in range(_C3):
                    chv = zero16 + ch
                    acc = wg[0] * plsc.load_gather(g_grid, [pt, chv])
                    for c in range(1, 8):
                        acc = acc + wg[c] * plsc.load_gather(
                            g_grid, [pt + c * _CH, chv])
                    emb[48 + ch, pl.ds(s, 16)] = acc
                return c_

            lax.fori_loop(0, _CH // 16, phase_sum, 0)
            pltpu.sync_copy(emb, out_hbm.at[wid * nch + ci])
            return carry

        lax.fori_loop(0, nch, chunk, 0)

    return sc_embed


def _mlp_body(ts_ref, emb_ref, m0p, mb0, m1h, m1l, mb1, m2h, m2l, mb2,
              w0r, b0r, w1r, b1r, w2r, b2r, wlp, blp, out_ref):
    hi = lax.Precision.HIGHEST

    def dot(a, b):
        return lax.dot(a, b, precision=hi, preferred_element_type=jnp.float32)

    lat = emb_ref[...]
    h0 = jnp.maximum(dot(m0p[...], lat) + mb0[...], 0.0)
    h1 = jnp.maximum(dot(m1h[...], h0) + dot(m1l[...], lat) + mb1[...], 0.0)
    h2 = jnp.maximum(dot(m2h[...], h1) + dot(m2l[...], lat) + mb2[...], 0.0)
    x = jnp.sin(30.0 * (w0r[...] * ts_ref[0] + b0r[...])) * h0
    x = jnp.sin(dot(w1r[...], x) + b1r[...]) * h1
    x = jnp.sin(dot(w2r[...], x) + b2r[...]) * h2
    out_ref[...] = dot(wlp[...], x) + blp[...]


def _mlp(ts3, emb, weights):
    grid, _, BN = ts3.shape
    N = grid * BN

    def full(a):
        return pl.BlockSpec(a.shape, lambda i: (0,) * a.ndim)

    return pl.pallas_call(
        _mlp_body,
        grid=(grid,),
        in_specs=[pl.BlockSpec((1, 1, BN), lambda i: (i, 0, 0)),
                  pl.BlockSpec((_EW, BN), lambda i: (0, i))] +
                 [full(a) for a in weights],
        out_specs=pl.BlockSpec((8, BN), lambda i: (0, i)),
        out_shape=jax.ShapeDtypeStruct((8, N), jnp.float32),
        compiler_params=pltpu.CompilerParams(dimension_semantics=("parallel",)),
    )(ts3, emb, *weights)


def kernel(temporal_steps, all_coords, table_xy, table_yt, table_xt, grid3d,
           W0, b0, W1, b1, W2, b2, Wl, bl, M0, Mb0, M1, Mb1, M2, Mb2):
    b, t = temporal_steps.shape
    N = b * t
    ac_t = all_coords.reshape(N, 3).T
    emb3 = _build_sc_embed(N)(
        ac_t,
        table_xy.reshape(_L * _T // 4, 8),
        table_yt.reshape(_L * _T // 4, 8),
        table_xt.reshape(_L * _T // 4, 8),
        jnp.pad(grid3d.reshape(-1, _C3), ((0, 0), (0, 12))),
    )
    emb = emb3.transpose(1, 0, 2).reshape(_EW, N)

    padc = ((0, 0), (0, _EW - 84))
    m0p = jnp.pad(M0, padc)
    m1h, m1l = M1[:, :64], jnp.pad(M1[:, 64:], padc)
    m2h, m2l = M2[:, :64], jnp.pad(M2[:, 64:], padc)
    wlp = jnp.pad(Wl, ((0, 5), (0, 0)))
    blp = jnp.pad(bl, (0, 5)).reshape(8, 1)
    col = lambda v: v.reshape(-1, 1)
    BN = 2048
    ts3 = temporal_steps.reshape(N // BN, 1, BN)
    weights = (m0p, col(Mb0), m1h, m1l, col(Mb1), m2h, m2l, col(Mb2),
               W0, col(b0), W1, col(b1), W2, col(b2), wlp, blp)
    out = _mlp(ts3, emb, weights)
    return out[:3].T.reshape(b, t, 3)
